# Initial kernel scaffold; baseline (speedup 1.0000x reference)
import jax
import jax.numpy as jnp
from jax.experimental import pallas as pl


def kernel(points, feats, W_sp, b_sp, theta_low, b_low, theta_high, b_high, W_g1, b_g1, ln_g1_gamma, ln_g1_beta, W_g2, b_g2, W_out, b_out, ln_out_gamma, ln_out_beta, gamma_res):
    raise NotImplementedError("write your pallas kernel here")


# speedup vs baseline: 11.6997x; 11.6997x over previous
import functools

import jax
import jax.numpy as jnp
from jax import lax
from jax.experimental import pallas as pl
from jax.experimental.pallas import tpu as pltpu
from jax.experimental.pallas import tpu_sc as plsc

B, N, C, KNN = 8, 2048, 128, 16
M = B * N
R = 256
NW = 32
PW = M // NW
P = 8


def _knn_body(pts_ref, ptst_ref, out_ref):
    b = pl.program_id(0)
    r = pl.program_id(1)
    pt = pts_ref[0]
    ptt = ptst_ref[0]
    sq_i = jnp.sum(pt * pt, axis=1, keepdims=True)
    sq_j = jnp.sum(ptt * ptt, axis=0, keepdims=True)
    g = lax.dot_general(pt, ptt, (((1,), (0,)), ((), ())),
                        preferred_element_type=jnp.float32)
    d = sq_i + sq_j - 2.0 * g
    ii = lax.broadcasted_iota(jnp.int32, (R, N), 0) + r * R
    jj = lax.broadcasted_iota(jnp.int32, (R, N), 1)
    d = jnp.where(jj == ii, 1e10, d)
    cols = []
    for _ in range(KNN):
        m = jnp.min(d, axis=1, keepdims=True)
        cand = jnp.where(d == m, jj, jnp.int32(2 * N))
        sel = jnp.min(cand, axis=1, keepdims=True)
        cols.append(sel)
        d = jnp.where(jj == sel, jnp.float32(1e30), d)
    idx = jnp.concatenate(cols, axis=1)
    out_ref[...] = idx + b * N


def _knn_indices(points, pts_t):
    return pl.pallas_call(
        _knn_body,
        grid=(B, N // R),
        in_specs=[
            pl.BlockSpec((1, R, 3), lambda b, r: (b, r, 0)),
            pl.BlockSpec((1, 3, N), lambda b, r: (b, 0, 0)),
        ],
        out_specs=pl.BlockSpec((R, KNN), lambda b, r: (b * (N // R) + r, 0)),
        out_shape=jax.ShapeDtypeStruct((M, KNN), jnp.int32),
    )(points, pts_t)


def _make_sc_agg(a_coef, b_coef, use_aux, c_coef=0.0):
    mesh = plsc.VectorSubcoreMesh(core_axis_name="c", subcore_axis_name="s")
    scratch = [
        pltpu.VMEM((P * KNN,), jnp.int32),
        pltpu.VMEM((P * KNN, C), jnp.float32),
        pltpu.VMEM((P, C), jnp.float32),
    ]
    if use_aux:
        scratch.append(pltpu.VMEM((P, C), jnp.float32))
    scratch += [pltpu.VMEM((P, C), jnp.float32), pltpu.SemaphoreType.DMA]

    def _body(*refs):
        if use_aux:
            (table_hbm, aux_hbm, gidx_hbm, out_hbm,
             idx_v, rows_v, x_v, aux_v, out_v, sem) = refs
        else:
            (table_hbm, gidx_hbm, out_hbm,
             idx_v, rows_v, x_v, out_v, sem) = refs
            aux_hbm = aux_v = None
        wid = lax.axis_index("s") * 2 + lax.axis_index("c")
        base = wid * PW

        def chunk(j, carry):
            goff = base + j * P
            pltpu.sync_copy(gidx_hbm.at[pl.ds(goff * KNN, P * KNN)], idx_v)
            cp = pltpu.async_copy(table_hbm.at[idx_v], rows_v, sem)
            pltpu.sync_copy(table_hbm.at[pl.ds(goff, P)], x_v)
            if use_aux:
                pltpu.sync_copy(aux_hbm.at[pl.ds(goff, P)], aux_v)
            cp.wait()
            for p in range(P):
                for c in range(C // 16):
                    sl = pl.ds(c * 16, 16)
                    acc = rows_v[p * KNN, sl]
                    for rr in range(1, KNN):
                        acc = acc + rows_v[p * KNN + rr, sl]
                    val = a_coef * x_v[p, sl] + b_coef * acc
                    if use_aux:
                        val = val + c_coef * aux_v[p, sl]
                    out_v[p, sl] = val
            pltpu.sync_copy(out_v, out_hbm.at[pl.ds(goff, P)])
            return carry

        lax.fori_loop(0, PW // P, chunk, 0)

    return functools.partial(
        pl.kernel, mesh=mesh,
        out_type=jax.ShapeDtypeStruct((M, C), jnp.float32),
        scratch_types=scratch,
    )(_body)


def _layernorm(x, g, b):
    mu = jnp.mean(x, axis=-1, keepdims=True)
    var = jnp.mean((x - mu) * (x - mu), axis=-1, keepdims=True)
    return (x - mu) / jnp.sqrt(var + 1e-5) * g + b


def _dense_body(x_ref, t1_ref, t2_ref, wsp_ref, bsp_ref, thl_ref, bl_ref,
                thh_ref, bh_ref, wg1_ref, bg1_ref, g1g_ref, g1b_ref,
                wg2_ref, bg2_ref, wout_ref, bout_ref, og_ref, ob_ref,
                gr_ref, out_ref):
    x = x_ref[...]
    t1 = t1_ref[...]
    t2 = t2_ref[...]

    def mm(a, w):
        return jnp.dot(a, w, preferred_element_type=jnp.float32)

    f_sp = mm(x, wsp_ref[...]) + bsp_ref[...]
    f_lo = mm(x, thl_ref[0]) + mm(t1, thl_ref[1]) + mm(t2, thl_ref[2]) + bl_ref[...]
    f_hi = mm(x, thh_ref[0]) + mm(t1, thh_ref[1]) + mm(t2, thh_ref[2]) + bh_ref[...]
    h = (mm(f_sp, wg1_ref[0]) + mm(f_lo, wg1_ref[1]) + mm(f_hi, wg1_ref[2])
         + bg1_ref[...])
    h = jax.nn.relu(_layernorm(h, g1g_ref[...], g1b_ref[...]))
    gate = mm(h, wg2_ref[...]) + bg2_ref[...]
    gate = gate - jnp.max(gate, axis=-1, keepdims=True)
    e = jnp.exp(gate)
    gate = e / jnp.sum(e, axis=-1, keepdims=True)
    f_fused = (gate[:, 0:1] * f_sp + gate[:, 1:2] * f_lo + gate[:, 2:3] * f_hi)
    out = mm(f_fused, wout_ref[...]) + bout_ref[...]
    out = _layernorm(out, og_ref[...], ob_ref[...])
    out_ref[...] = x + gr_ref[0, 0] * out


def _dense_tail(x, t1, t2, wsp, bsp, thl, bl, thh, bh, wg1, bg1, g1g, g1b,
                wg2, bg2, wout, bout, og, ob, gr):
    row = lambda t: (t, 0)
    full2 = lambda t: (0, 0)
    full3 = lambda t: (0, 0, 0)
    specs = [
        pl.BlockSpec((R, C), row),
        pl.BlockSpec((R, C), row),
        pl.BlockSpec((R, C), row),
        pl.BlockSpec((C, C), full2),
        pl.BlockSpec((1, C), full2),
        pl.BlockSpec((3, C, C), full3),
        pl.BlockSpec((1, C), full2),
        pl.BlockSpec((3, C, C), full3),
        pl.BlockSpec((1, C), full2),
        pl.BlockSpec((3, C, C), full3),
        pl.BlockSpec((1, C), full2),
        pl.BlockSpec((1, C), full2),
        pl.BlockSpec((1, C), full2),
        pl.BlockSpec((C, 3), full2),
        pl.BlockSpec((1, 3), full2),
        pl.BlockSpec((C, C), full2),
        pl.BlockSpec((1, C), full2),
        pl.BlockSpec((1, C), full2),
        pl.BlockSpec((1, C), full2),
        pl.BlockSpec((1, 1), full2),
    ]
    return pl.pallas_call(
        _dense_body,
        grid=(M // R,),
        in_specs=specs,
        out_specs=pl.BlockSpec((R, C), row),
        out_shape=jax.ShapeDtypeStruct((M, C), jnp.float32),
    )(x, t1, t2, wsp, bsp, thl, bl, thh, bh, wg1, bg1, g1g, g1b,
      wg2, bg2, wout, bout, og, ob, gr)


@functools.lru_cache(maxsize=None)
def _sc_aggs():
    return (_make_sc_agg(1.0, -1.0 / KNN, use_aux=False),
            _make_sc_agg(2.0, -2.0 / KNN, use_aux=True, c_coef=-1.0))


def kernel(points, feats, W_sp, b_sp, theta_low, b_low, theta_high, b_high,
           W_g1, b_g1, ln_g1_gamma, ln_g1_beta, W_g2, b_g2, W_out, b_out,
           ln_out_gamma, ln_out_beta, gamma_res):
    pts_t = jnp.swapaxes(points, 1, 2)
    gidx = _knn_indices(points, pts_t).reshape(-1)
    xf = feats.reshape(M, C)
    sc_t1, sc_t2 = _sc_aggs()
    t1 = sc_t1(xf, gidx)
    t2 = sc_t2(t1, xf, gidx)
    r2 = lambda v: v.reshape(1, -1)
    out = _dense_tail(
        xf, t1, t2, W_sp, r2(b_sp), theta_low, r2(b_low), theta_high,
        r2(b_high), W_g1.reshape(3, C, C), r2(b_g1), r2(ln_g1_gamma),
        r2(ln_g1_beta), W_g2, r2(b_g2), W_out, r2(b_out), r2(ln_out_gamma),
        r2(ln_out_beta), gamma_res.reshape(1, 1))
    return out.reshape(B, N, C)


# speedup vs baseline: 21.2217x; 1.8139x over previous
import functools

import jax
import jax.numpy as jnp
from jax import lax
from jax.experimental import pallas as pl
from jax.experimental.pallas import tpu as pltpu
from jax.experimental.pallas import tpu_sc as plsc

B, N, C, KNN = 8, 2048, 128, 16
M = B * N
R = 256
NW = 32
PW = M // NW
P = 8


def _knn_body(pts_ref, ptst_ref, out_ref):
    b = pl.program_id(0)
    r = pl.program_id(1)
    pt = pts_ref[0]
    ptt = ptst_ref[0]
    sq_i = jnp.sum(pt * pt, axis=1, keepdims=True)
    sq_j = jnp.sum(ptt * ptt, axis=0, keepdims=True)
    g = lax.dot_general(pt, ptt, (((1,), (0,)), ((), ())),
                        preferred_element_type=jnp.float32)
    d = sq_i + sq_j - 2.0 * g
    ii = lax.broadcasted_iota(jnp.int32, (R, N), 0) + r * R
    jj = lax.broadcasted_iota(jnp.int32, (R, N), 1)
    d = jnp.where(jj == ii, 1e10, d)
    kb = lax.bitcast_convert_type(d, jnp.int32)
    kb = (kb & jnp.int32(~(N - 1))) | jj
    kf = lax.bitcast_convert_type(kb, jnp.float32)
    cols = []
    for _ in range(KNN):
        m = jnp.min(kf, axis=1, keepdims=True)
        cols.append(lax.bitcast_convert_type(m, jnp.int32) & jnp.int32(N - 1))
        kf = jnp.where(kf == m, jnp.float32(3e38), kf)
    idx = jnp.concatenate(cols, axis=1)
    out_ref[...] = idx + b * N


def _knn_indices(points, pts_t):
    return pl.pallas_call(
        _knn_body,
        grid=(B, N // R),
        in_specs=[
            pl.BlockSpec((1, R, 3), lambda b, r: (b, r, 0)),
            pl.BlockSpec((1, 3, N), lambda b, r: (b, 0, 0)),
        ],
        out_specs=pl.BlockSpec((R, KNN), lambda b, r: (b * (N // R) + r, 0)),
        out_shape=jax.ShapeDtypeStruct((M, KNN), jnp.int32),
    )(points, pts_t)


def _make_sc_agg():
    mesh = plsc.VectorSubcoreMesh(core_axis_name="c", subcore_axis_name="s")
    scratch = [
        pltpu.VMEM((P * KNN,), jnp.int32),
        pltpu.VMEM((P * KNN,), jnp.int32),
        pltpu.VMEM((P * KNN, C), jnp.float32),
        pltpu.VMEM((P * KNN, C), jnp.float32),
        pltpu.VMEM((PW, C), jnp.float32),
        pltpu.VMEM((P, C), jnp.float32),
        pltpu.VMEM((P, C), jnp.float32),
        pltpu.SemaphoreType.DMA,
        pltpu.SemaphoreType.DMA,
        pltpu.SemaphoreType.DMA,
        pltpu.SemaphoreType.DMA,
    ]
    NCH = PW // P
    NPAIR = NCH // 2
    scale = -1.0 / KNN

    def _body(table_hbm, gidx_hbm, out_hbm, idx0, idx1, rows0, rows1,
              xs, out0, out1, sg0, sg1, so0, so1):
        wid = lax.axis_index("s") * 2 + lax.axis_index("c")
        base = wid * PW

        def accumulate(c, rows_v, out_v):
            for p in range(P):
                for ch in range(C // 16):
                    sl = pl.ds(ch * 16, 16)
                    s = []
                    for rr in range(0, KNN, 2):
                        s.append(rows_v[p * KNN + rr, sl]
                                 + rows_v[p * KNN + rr + 1, sl])
                    while len(s) > 1:
                        s = [a + b for a, b in
                             zip(s[::2], s[1::2])] + ([s[-1]] if len(s) % 2 else [])
                    out_v[p, sl] = xs[c * P + p, sl] + scale * s[0]

        def fetch_idx_and_gather(c, idx_v, rows_v, sem):
            pltpu.sync_copy(gidx_hbm.at[pl.ds((base + c * P) * KNN, P * KNN)],
                            idx_v)
            pltpu.async_copy(table_hbm.at[idx_v], rows_v, sem)

        def wait_gather(idx_v, rows_v, sem):
            pltpu.make_async_copy(table_hbm.at[idx_v], rows_v, sem).wait()

        def put_out(c, out_v, sem):
            pltpu.async_copy(out_v, out_hbm.at[pl.ds(base + c * P, P)], sem)

        def wait_out(c, out_v, sem):
            pltpu.make_async_copy(out_v, out_hbm.at[pl.ds(base + c * P, P)],
                                  sem).wait()

        pltpu.sync_copy(table_hbm.at[pl.ds(base, PW)], xs)
        fetch_idx_and_gather(0, idx0, rows0, sg0)

        def pair(pr, carry):
            c0 = 2 * pr
            c1 = c0 + 1
            fetch_idx_and_gather(c1, idx1, rows1, sg1)
            wait_gather(idx0, rows0, sg0)

            @pl.when(pr > 0)
            def _():
                wait_out(c0 - 2, out0, so0)
            accumulate(c0, rows0, out0)
            put_out(c0, out0, so0)

            @pl.when(pr < NPAIR - 1)
            def _():
                fetch_idx_and_gather(c0 + 2, idx0, rows0, sg0)
            wait_gather(idx1, rows1, sg1)

            @pl.when(pr > 0)
            def _():
                wait_out(c1 - 2, out1, so1)
            accumulate(c1, rows1, out1)
            put_out(c1, out1, so1)
            return carry

        lax.fori_loop(0, NPAIR, pair, 0)
        wait_out(NCH - 2, out0, so0)
        wait_out(NCH - 1, out1, so1)

    return functools.partial(
        pl.kernel, mesh=mesh,
        out_type=jax.ShapeDtypeStruct((M, C), jnp.float32),
        scratch_types=scratch,
    )(_body)


def _layernorm(x, g, b):
    mu = jnp.mean(x, axis=-1, keepdims=True)
    var = jnp.mean((x - mu) * (x - mu), axis=-1, keepdims=True)
    return (x - mu) / jnp.sqrt(var + 1e-5) * g + b


def _dense_body(x_ref, t1_ref, lt1_ref, wsp_ref, bsp_ref, thl_ref, bl_ref,
                thh_ref, bh_ref, wg1_ref, bg1_ref, g1g_ref, g1b_ref,
                wg2_ref, bg2_ref, wout_ref, bout_ref, og_ref, ob_ref,
                gr_ref, out_ref):
    x = x_ref[...]
    t1 = t1_ref[...]
    t2 = 2.0 * lt1_ref[...] - x

    def mm(a, w):
        return jnp.dot(a, w, preferred_element_type=jnp.float32)

    f_sp = mm(x, wsp_ref[...]) + bsp_ref[...]
    f_lo = mm(x, thl_ref[0]) + mm(t1, thl_ref[1]) + mm(t2, thl_ref[2]) + bl_ref[...]
    f_hi = mm(x, thh_ref[0]) + mm(t1, thh_ref[1]) + mm(t2, thh_ref[2]) + bh_ref[...]
    h = (mm(f_sp, wg1_ref[0]) + mm(f_lo, wg1_ref[1]) + mm(f_hi, wg1_ref[2])
         + bg1_ref[...])
    h = jax.nn.relu(_layernorm(h, g1g_ref[...], g1b_ref[...]))
    gate = mm(h, wg2_ref[...]) + bg2_ref[...]
    gate = gate - jnp.max(gate, axis=-1, keepdims=True)
    e = jnp.exp(gate)
    gate = e / jnp.sum(e, axis=-1, keepdims=True)
    f_fused = (gate[:, 0:1] * f_sp + gate[:, 1:2] * f_lo + gate[:, 2:3] * f_hi)
    out = mm(f_fused, wout_ref[...]) + bout_ref[...]
    out = _layernorm(out, og_ref[...], ob_ref[...])
    out_ref[...] = x + gr_ref[0, 0] * out


def _dense_tail(x, t1, lt1, wsp, bsp, thl, bl, thh, bh, wg1, bg1, g1g, g1b,
                wg2, bg2, wout, bout, og, ob, gr):
    row = lambda t: (t, 0)
    full2 = lambda t: (0, 0)
    full3 = lambda t: (0, 0, 0)
    specs = [
        pl.BlockSpec((R, C), row),
        pl.BlockSpec((R, C), row),
        pl.BlockSpec((R, C), row),
        pl.BlockSpec((C, C), full2),
        pl.BlockSpec((1, C), full2),
        pl.BlockSpec((3, C, C), full3),
        pl.BlockSpec((1, C), full2),
        pl.BlockSpec((3, C, C), full3),
        pl.BlockSpec((1, C), full2),
        pl.BlockSpec((3, C, C), full3),
        pl.BlockSpec((1, C), full2),
        pl.BlockSpec((1, C), full2),
        pl.BlockSpec((1, C), full2),
        pl.BlockSpec((C, 3), full2),
        pl.BlockSpec((1, 3), full2),
        pl.BlockSpec((C, C), full2),
        pl.BlockSpec((1, C), full2),
        pl.BlockSpec((1, C), full2),
        pl.BlockSpec((1, C), full2),
        pl.BlockSpec((1, 1), full2),
    ]
    return pl.pallas_call(
        _dense_body,
        grid=(M // R,),
        in_specs=specs,
        out_specs=pl.BlockSpec((R, C), row),
        out_shape=jax.ShapeDtypeStruct((M, C), jnp.float32),
    )(x, t1, lt1, wsp, bsp, thl, bl, thh, bh, wg1, bg1, g1g, g1b,
      wg2, bg2, wout, bout, og, ob, gr)


@functools.lru_cache(maxsize=None)
def _sc_agg():
    return _make_sc_agg()


def kernel(points, feats, W_sp, b_sp, theta_low, b_low, theta_high, b_high,
           W_g1, b_g1, ln_g1_gamma, ln_g1_beta, W_g2, b_g2, W_out, b_out,
           ln_out_gamma, ln_out_beta, gamma_res):
    pts_t = jnp.swapaxes(points, 1, 2)
    gidx = _knn_indices(points, pts_t).reshape(-1)
    xf = feats.reshape(M, C)
    sc_l = _sc_agg()
    t1 = sc_l(xf, gidx)
    lt1 = sc_l(t1, gidx)
    r2 = lambda v: v.reshape(1, -1)
    out = _dense_tail(
        xf, t1, lt1, W_sp, r2(b_sp), theta_low, r2(b_low), theta_high,
        r2(b_high), W_g1.reshape(3, C, C), r2(b_g1), r2(ln_g1_gamma),
        r2(ln_g1_beta), W_g2, r2(b_g2), W_out, r2(b_out), r2(ln_out_gamma),
        r2(ln_out_beta), gamma_res.reshape(1, 1))
    return out.reshape(B, N, C)
